# Initial kernel scaffold; baseline (speedup 1.0000x reference)
import jax
import jax.numpy as jnp
from jax.experimental import pallas as pl


def kernel(pos_embs, neg_embs, neut_embs, assas_emb, vocab_table, W1, b1, W2, b2, W3, b3, W4, b4):
    raise NotImplementedError("write your pallas kernel here")


# speedup vs baseline: 3.2708x; 3.2708x over previous
import functools
import jax
import jax.numpy as jnp
from jax import lax
from jax.experimental import pallas as pl
from jax.experimental.pallas import tpu as pltpu
from jax.experimental.pallas import tpu_sc as plsc

_B = 64
_D = 768
_NPOS, _NNEG, _NNEUT = 9, 8, 7
_V = 100000
_VS = 80
_K = 40

_VT = 1024
_NT = 98
_VPAD = _VT * _NT


def _layer(x, w_ref, b_ref, act):
    y = lax.dot_general(x, w_ref[...], (((1,), (1,)), ((), ())),
                        preferred_element_type=jnp.float32) + b_ref[...]
    return jnp.tanh(y) if act else y


def _mlp_a_body(cat_ref, w1_ref, b1_ref, w2_ref, b2_ref, h_ref):
    h = _layer(cat_ref[...], w1_ref, b1_ref, True)
    h_ref[...] = _layer(h, w2_ref, b2_ref, True)


def _mlp_b_body(h_ref, w3_ref, b3_ref, w4_ref, b4_ref, outn_ref):
    h = _layer(h_ref[...], w3_ref, b3_ref, True)
    mo = _layer(h, w4_ref, b4_ref, False)
    n = jnp.sqrt(jnp.sum(mo * mo, axis=1, keepdims=True))
    outn_ref[...] = mo / jnp.maximum(n, 1e-12)


def _run_mlp(pos, neg, neut, ass, W1, b1, W2, b2, W3, b3, W4, b4):
    def proc(x):
        m = jnp.mean(x, axis=1)
        n = jnp.sqrt(jnp.sum(m * m, axis=1, keepdims=True))
        return m / jnp.maximum(n, 1e-12)

    cat = jnp.concatenate([proc(neg), ass, proc(neut), proc(pos)], axis=1)
    h = pl.pallas_call(
        _mlp_a_body,
        out_shape=jax.ShapeDtypeStruct((_B, 1700), jnp.float32),
    )(cat, W1, b1[None, :], W2, b2[None, :])
    return pl.pallas_call(
        _mlp_b_body,
        out_shape=jax.ShapeDtypeStruct((_B, _D), jnp.float32),
    )(h, W3, b3[None, :], W4, b4[None, :])


def _sims_body(mon_ref, vocab_ref, sims_ref, bm_ref):
    i = pl.program_id(0)
    v = vocab_ref[...]
    ss = jnp.sum(v * v, axis=1, keepdims=True)
    norm = jnp.maximum(jnp.sqrt(ss), 1e-12)
    vn = v / norm
    s = lax.dot_general(mon_ref[...], vn, (((1,), (1,)), ((), ())),
                        preferred_element_type=jnp.float32)
    col = i * _VT + lax.broadcasted_iota(jnp.int32, (_B, _VT), 1)
    s = jnp.where(col < _V, s, -jnp.inf)
    sims_ref[...] = s
    bm_ref[...] = jnp.max(s.reshape(_B, _VT // 128, 128), axis=2)[None]


def _run_sims(mon, vocab):
    return pl.pallas_call(
        _sims_body,
        grid=(_NT,),
        in_specs=[
            pl.BlockSpec((_B, _D), lambda i: (0, 0)),
            pl.BlockSpec((_VT, _D), lambda i: (i, 0)),
        ],
        out_specs=(pl.BlockSpec((_B, _VT), lambda i: (0, i)),
                   pl.BlockSpec((1, _B, _VT // 128), lambda i: (i, 0, 0))),
        out_shape=(jax.ShapeDtypeStruct((_B, _VPAD), jnp.float32),
                   jax.ShapeDtypeStruct((_NT, _B, _VT // 128), jnp.float32)),
    )(mon, vocab)


def _thr_body(bm_ref, thr_ref):
    bm = bm_ref[...]
    ib = lax.bitcast_convert_type(bm, jnp.int32)
    keys = jnp.where(ib < 0, ib ^ jnp.int32(0x7FFFFFFF), ib)

    def sb(_, lh):
        lo, hi = lh
        mid = (lo >> 1) + (hi >> 1) + (lo & hi & 1)
        cnt = jnp.sum((keys >= mid).astype(jnp.int32), axis=1, keepdims=True)
        big = cnt >= _VS
        return jnp.where(big, mid, lo), jnp.where(big, hi, mid)

    lo0 = jnp.full((_B, 1), _KEY_LO, jnp.int32)
    hi0 = jnp.full((_B, 1), _KEY_HI, jnp.int32)
    lo, _hi = lax.fori_loop(0, 32, sb, (lo0, hi0))
    thr = lax.bitcast_convert_type(
        jnp.where(lo < 0, lo ^ jnp.int32(0x7FFFFFFF), lo), jnp.float32)
    thr_ref[...] = jnp.broadcast_to(thr, (_B, 16))


def _run_thr(bm):
    return pl.pallas_call(
        _thr_body,
        out_shape=jax.ShapeDtypeStruct((_B, 16), jnp.float32),
    )(bm)


_CH = 12544
_NCH = _VPAD // _CH
_NVR = _CH // 16
_CAP = 1024
_KEY_LO = -2139095042
_KEY_HI = 2139095041


def _sc_topk_body(sims_hbm, thr_hbm, vocab_hbm, words_hbm,
                  buf, tbuf, candv, candi, oidx, gbuf, sem):
    iota = lax.broadcasted_iota(jnp.int32, (16,), 0)
    wid = lax.axis_index("c") * 16 + lax.axis_index("s")

    def row_body(rr, _):
        row = wid * 2 + rr
        base = row * _VPAD
        pltpu.sync_copy(thr_hbm.at[pl.ds(row * 16, 16)], tbuf)
        t = tbuf[pl.ds(0, 16)]

        def chunk_body(k, c):
            pltpu.sync_copy(sims_hbm.at[pl.ds(base + k * _CH, _CH)], buf)

            def vreg_body(g, c2):
                v = buf[pl.ds(g * 16, 16)]
                m = v >= t
                off = jnp.minimum(c2, _CAP - 16)
                plsc.store_compressed(candv.at[pl.ds(off, 16)], v, mask=m)
                plsc.store_compressed(candi.at[pl.ds(off, 16)],
                                      iota + (k * _CH + g * 16), mask=m)
                return jnp.minimum(c2 + jnp.sum(jnp.where(m, 1, 0)),
                                   jnp.int32(_CAP - 16))
            return lax.fori_loop(0, _NVR, vreg_body, c)

        cf = lax.fori_loop(0, _NCH, chunk_body, jnp.int32(0))

        def oinit(j, _):
            oidx[pl.ds(j * 16, 16)] = iota * 0
            return 0
        lax.fori_loop(0, 6, oinit, 0)
        nvr = (cf + 15) >> 4

        def rank_body(i, _):
            vi = candv[pl.ds(i * 16, 16)]
            ii = candi[pl.ds(i * 16, 16)]
            valid_i = (iota + i * 16) < cf

            def rloop(j, rv):
                js = iota * 0 + j
                a = plsc.load_gather(candv, [js])
                ai = plsc.load_gather(candi, [js])
                beats = (a > vi) | ((a == vi) & (ai < ii))
                return rv + jnp.where(beats, 1, 0)
            rv = lax.fori_loop(0, cf, rloop, jnp.zeros((16,), jnp.int32))
            ok = valid_i & (rv < _VS)
            rv = jnp.where(ok, rv, _VS + iota)
            plsc.store_scatter(oidx, [rv], jnp.where(ok, ii, 0))
            return 0
        lax.fori_loop(0, nvr, rank_body, 0)

        pltpu.async_copy(vocab_hbm.at[oidx], gbuf, sem).wait()
        pltpu.sync_copy(gbuf.at[pl.ds(0, _VS)], words_hbm.at[row])
        return 0

    lax.fori_loop(0, 2, row_body, 0)


def _run_sc_topk(sims, thr, vocab):
    mesh = plsc.VectorSubcoreMesh(core_axis_name="c", subcore_axis_name="s")
    f = pl.kernel(
        _sc_topk_body,
        out_type=jax.ShapeDtypeStruct((_B, _VS, _D), jnp.float32),
        mesh=mesh,
        compiler_params=pltpu.CompilerParams(needs_layout_passes=False),
        scratch_types=[
            pltpu.VMEM((_CH,), jnp.float32),
            pltpu.VMEM((16,), jnp.float32),
            pltpu.VMEM((_CAP,), jnp.float32),
            pltpu.VMEM((_CAP,), jnp.int32),
            pltpu.VMEM((_VS + 16,), jnp.int32),
            pltpu.VMEM((_VS + 16, _D), jnp.float32),
            pltpu.SemaphoreType.DMA,
        ],
    )
    return f(sims.reshape(-1), thr.reshape(-1), vocab)


def _reward_body(w_ref, pos_ref, neg_ref, neut_ref, ass_ref,
                 se_ref, mx_ref, mn_ref):
    w = w_ref[0]

    def nrm(x):
        n = jnp.sqrt(jnp.sum(x * x, axis=1, keepdims=True))
        return x / jnp.maximum(n, 1e-12)

    wn = nrm(w)
    pn = nrm(pos_ref[0])
    ngn = nrm(neg_ref[0])
    ntn = nrm(neut_ref[0])
    an = nrm(ass_ref[0])

    def score(e):
        return lax.dot_general(wn, e, (((1,), (1,)), ((), ())),
                               preferred_element_type=jnp.float32,
                               precision=lax.Precision.HIGHEST)

    s_pos = score(pn)
    s_neg = score(ngn)
    s_neut = score(ntn)
    s_ass = score(an)

    m_neg = jnp.max(s_neg, axis=1, keepdims=True)
    m_neut = jnp.max(s_neut, axis=1, keepdims=True)
    m_ass = s_ass
    m_np = jnp.maximum(jnp.maximum(m_neg, m_neut), m_ass)

    primary = jnp.sum((s_pos >= m_np).astype(jnp.float32), axis=1,
                      keepdims=True)
    secondary = jnp.where(m_neg >= m_np, 0.0,
                          jnp.where(m_neut >= m_np, 1.0, -10.0))
    tot = primary + secondary

    iota_c = lax.broadcasted_iota(jnp.int32, (_VS, 1), 0).astype(jnp.float32)
    key_c = tot * 128.0 - iota_c
    key_r = jnp.transpose(key_c)
    gt = (key_r > key_c).astype(jnp.float32)
    rank = jnp.sum(gt, axis=1, keepdims=True)
    mask_max = (rank < _K).astype(jnp.float32)
    key2_c = -tot * 128.0 - iota_c
    key2_r = jnp.transpose(key2_c)
    gt2 = (key2_r > key2_c).astype(jnp.float32)
    rank2 = jnp.sum(gt2, axis=1, keepdims=True)
    mask_min = (rank2 < _K).astype(jnp.float32)
    onehot = (key_c >= jnp.max(key_c)).astype(jnp.float32)

    def pool(mask):
        s = jnp.sum(w * mask, axis=0, keepdims=True) / float(_K)
        n = jnp.sqrt(jnp.sum(s * s, axis=1, keepdims=True))
        return s / jnp.maximum(n, 1e-12)

    se_ref[...] = jnp.sum(w * onehot, axis=0, keepdims=True)[None]
    mx_ref[...] = pool(mask_max)[None]
    mn_ref[...] = pool(mask_min)[None]


def _run_reward(words, pos, neg, neut, ass):
    out_shape = tuple(jax.ShapeDtypeStruct((_B, 1, _D), jnp.float32)
                      for _ in range(3))
    spec_bd = pl.BlockSpec((1, 1, _D), lambda b: (b, 0, 0))
    se, mx, mn = pl.pallas_call(
        _reward_body,
        grid=(_B,),
        in_specs=[
            pl.BlockSpec((1, _VS, _D), lambda b: (b, 0, 0)),
            pl.BlockSpec((1, _NPOS, _D), lambda b: (b, 0, 0)),
            pl.BlockSpec((1, _NNEG, _D), lambda b: (b, 0, 0)),
            pl.BlockSpec((1, _NNEUT, _D), lambda b: (b, 0, 0)),
            spec_bd,
        ],
        out_specs=(spec_bd, spec_bd, spec_bd),
        out_shape=out_shape,
    )(words, pos, neg, neut, ass[:, None, :])
    return se[:, 0], mx[:, 0], mn[:, 0]


def kernel(pos_embs, neg_embs, neut_embs, assas_emb, vocab_table,
           W1, b1, W2, b2, W3, b3, W4, b4):
    mon = _run_mlp(pos_embs, neg_embs, neut_embs, assas_emb,
                   W1, b1, W2, b2, W3, b3, W4, b4)
    sims, bm3 = _run_sims(mon, vocab_table)
    bm = bm3.transpose(1, 0, 2).reshape(_B, _VPAD // 128)
    thr = _run_thr(bm)
    words = _run_sc_topk(sims, thr, vocab_table)
    se, mx, mn = _run_reward(words, pos_embs, neg_embs, neut_embs, assas_emb)
    return mon, se, mx, mn


# speedup vs baseline: 3.7904x; 1.1588x over previous
import functools
import jax
import jax.numpy as jnp
from jax import lax
from jax.experimental import pallas as pl
from jax.experimental.pallas import tpu as pltpu
from jax.experimental.pallas import tpu_sc as plsc

_B = 64
_D = 768
_NPOS, _NNEG, _NNEUT = 9, 8, 7
_V = 100000
_VS = 80
_K = 40

_VT = 1024
_NT = 98
_VPAD = _VT * _NT


def _layer(x, w_ref, b_ref, act):
    y = lax.dot_general(x, w_ref[...], (((1,), (1,)), ((), ())),
                        preferred_element_type=jnp.float32) + b_ref[...]
    return jnp.tanh(y) if act else y


def _mlp_a_body(cat_ref, w1_ref, b1_ref, w2_ref, b2_ref, h_ref):
    h = _layer(cat_ref[...], w1_ref, b1_ref, True)
    h_ref[...] = _layer(h, w2_ref, b2_ref, True)


def _mlp_b_body(h_ref, w3_ref, b3_ref, w4_ref, b4_ref, outn_ref):
    h = _layer(h_ref[...], w3_ref, b3_ref, True)
    mo = _layer(h, w4_ref, b4_ref, False)
    n = jnp.sqrt(jnp.sum(mo * mo, axis=1, keepdims=True))
    outn_ref[...] = mo / jnp.maximum(n, 1e-12)


def _run_mlp(pos, neg, neut, ass, W1, b1, W2, b2, W3, b3, W4, b4):
    def proc(x):
        m = jnp.mean(x, axis=1)
        n = jnp.sqrt(jnp.sum(m * m, axis=1, keepdims=True))
        return m / jnp.maximum(n, 1e-12)

    cat = jnp.concatenate([proc(neg), ass, proc(neut), proc(pos)], axis=1)
    h = pl.pallas_call(
        _mlp_a_body,
        out_shape=jax.ShapeDtypeStruct((_B, 1700), jnp.float32),
    )(cat, W1, b1[None, :], W2, b2[None, :])
    return pl.pallas_call(
        _mlp_b_body,
        out_shape=jax.ShapeDtypeStruct((_B, _D), jnp.float32),
    )(h, W3, b3[None, :], W4, b4[None, :])


def _sims_body(mon_ref, vocab_ref, sims_ref, bm_ref):
    i = pl.program_id(0)
    v = vocab_ref[...]
    ss = jnp.sum(v * v, axis=1, keepdims=True)
    norm = jnp.maximum(jnp.sqrt(ss), 1e-12)
    vn = v / norm
    s = lax.dot_general(mon_ref[...], vn, (((1,), (1,)), ((), ())),
                        preferred_element_type=jnp.float32)
    col = i * _VT + lax.broadcasted_iota(jnp.int32, (_B, _VT), 1)
    s = jnp.where(col < _V, s, -jnp.inf)
    sims_ref[...] = s
    bm_ref[...] = jnp.max(s.reshape(_B, _VT // 128, 128), axis=2)[None]


def _run_sims(mon, vocab):
    return pl.pallas_call(
        _sims_body,
        grid=(_NT,),
        in_specs=[
            pl.BlockSpec((_B, _D), lambda i: (0, 0)),
            pl.BlockSpec((_VT, _D), lambda i: (i, 0)),
        ],
        out_specs=(pl.BlockSpec((_B, _VT), lambda i: (0, i)),
                   pl.BlockSpec((1, _B, _VT // 128), lambda i: (i, 0, 0))),
        out_shape=(jax.ShapeDtypeStruct((_B, _VPAD), jnp.float32),
                   jax.ShapeDtypeStruct((_NT, _B, _VT // 128), jnp.float32)),
    )(mon, vocab)


def _thr_body(bm_ref, thr_ref):
    bm = bm_ref[...]
    ib = lax.bitcast_convert_type(bm, jnp.int32)
    keys = jnp.where(ib < 0, ib ^ jnp.int32(0x7FFFFFFF), ib)

    def sb(_, lh):
        lo, hi = lh
        mid = (lo >> 1) + (hi >> 1) + (lo & hi & 1)
        cnt = jnp.sum((keys >= mid).astype(jnp.int32), axis=1, keepdims=True)
        big = cnt >= _VS
        return jnp.where(big, mid, lo), jnp.where(big, hi, mid)

    lo0 = jnp.full((_B, 1), _KEY_LO, jnp.int32)
    hi0 = jnp.full((_B, 1), _KEY_HI, jnp.int32)
    lo, _hi = lax.fori_loop(0, 32, sb, (lo0, hi0))
    thr = lax.bitcast_convert_type(
        jnp.where(lo < 0, lo ^ jnp.int32(0x7FFFFFFF), lo), jnp.float32)
    thr_ref[...] = jnp.broadcast_to(thr, (_B, 16))


def _run_thr(bm):
    return pl.pallas_call(
        _thr_body,
        out_shape=jax.ShapeDtypeStruct((_B, 16), jnp.float32),
    )(bm)


_CH = 12544
_NCH = _VPAD // _CH
_NVR = _CH // 16
_CAP = 1024
_KEY_LO = -2139095042
_KEY_HI = 2139095041


def _sc_topk_body(sims_hbm, thr_hbm, vocab_hbm, words_hbm,
                  buf, tbuf, candv, candi, oidx, gbuf, sem):
    iota = lax.broadcasted_iota(jnp.int32, (16,), 0)
    wid = lax.axis_index("c") * 16 + lax.axis_index("s")

    def row_body(rr, _):
        row = wid * 2 + rr
        base = row * _VPAD
        pltpu.sync_copy(thr_hbm.at[pl.ds(row * 16, 16)], tbuf)
        t = tbuf[pl.ds(0, 16)]

        def chunk_body(k, c):
            pltpu.sync_copy(sims_hbm.at[pl.ds(base + k * _CH, _CH)], buf)

            def group_body(g, c2):
                vs = [buf[pl.ds(g * 128 + 16 * i, 16)] for i in range(8)]
                m01 = jnp.maximum(vs[0], vs[1])
                m23 = jnp.maximum(vs[2], vs[3])
                m45 = jnp.maximum(vs[4], vs[5])
                m67 = jnp.maximum(vs[6], vs[7])
                m8 = jnp.maximum(jnp.maximum(m01, m23), jnp.maximum(m45, m67))
                nhit = jnp.sum(jnp.where(m8 >= t, 1, 0))

                def slow(c3):
                    for i in range(8):
                        m = vs[i] >= t
                        off = jnp.minimum(c3, _CAP - 16)
                        plsc.store_compressed(candv.at[pl.ds(off, 16)], vs[i],
                                              mask=m)
                        plsc.store_compressed(candi.at[pl.ds(off, 16)],
                                              iota + (k * _CH + g * 128 + 16 * i),
                                              mask=m)
                        c3 = jnp.minimum(c3 + jnp.sum(jnp.where(m, 1, 0)),
                                         jnp.int32(_CAP - 16))
                    return c3
                return lax.cond(nhit > 0, slow, lambda x: x, c2)
            return lax.fori_loop(0, _NVR // 8, group_body, c)

        cf = lax.fori_loop(0, _NCH, chunk_body, jnp.int32(0))

        def oinit(j, _):
            oidx[pl.ds(j * 16, 16)] = iota * 0
            return 0
        lax.fori_loop(0, 6, oinit, 0)
        nvr = (cf + 15) >> 4

        def rank_body(i, _):
            vi = candv[pl.ds(i * 16, 16)]
            ii = candi[pl.ds(i * 16, 16)]
            valid_i = (iota + i * 16) < cf

            def rloop(j, rv):
                js = iota * 0 + j
                a = plsc.load_gather(candv, [js])
                ai = plsc.load_gather(candi, [js])
                beats = (a > vi) | ((a == vi) & (ai < ii))
                return rv + jnp.where(beats, 1, 0)
            rv = lax.fori_loop(0, cf, rloop, jnp.zeros((16,), jnp.int32))
            ok = valid_i & (rv < _VS)
            rv = jnp.where(ok, rv, _VS + iota)
            plsc.store_scatter(oidx, [rv], jnp.where(ok, ii, 0))
            return 0
        lax.fori_loop(0, nvr, rank_body, 0)

        pltpu.async_copy(vocab_hbm.at[oidx], gbuf, sem).wait()
        pltpu.sync_copy(gbuf.at[pl.ds(0, _VS)], words_hbm.at[row])
        return 0

    lax.fori_loop(0, 2, row_body, 0)


def _run_sc_topk(sims, thr, vocab):
    mesh = plsc.VectorSubcoreMesh(core_axis_name="c", subcore_axis_name="s")
    f = pl.kernel(
        _sc_topk_body,
        out_type=jax.ShapeDtypeStruct((_B, _VS, _D), jnp.float32),
        mesh=mesh,
        compiler_params=pltpu.CompilerParams(needs_layout_passes=False),
        scratch_types=[
            pltpu.VMEM((_CH,), jnp.float32),
            pltpu.VMEM((16,), jnp.float32),
            pltpu.VMEM((_CAP,), jnp.float32),
            pltpu.VMEM((_CAP,), jnp.int32),
            pltpu.VMEM((_VS + 16,), jnp.int32),
            pltpu.VMEM((_VS + 16, _D), jnp.float32),
            pltpu.SemaphoreType.DMA,
        ],
    )
    return f(sims.reshape(-1), thr.reshape(-1), vocab)


def _reward_body(w_ref, pos_ref, neg_ref, neut_ref, ass_ref,
                 se_ref, mx_ref, mn_ref):
    w = w_ref[0]

    def nrm(x):
        n = jnp.sqrt(jnp.sum(x * x, axis=1, keepdims=True))
        return x / jnp.maximum(n, 1e-12)

    wn = nrm(w)
    pn = nrm(pos_ref[0])
    ngn = nrm(neg_ref[0])
    ntn = nrm(neut_ref[0])
    an = nrm(ass_ref[0])

    def score(e):
        return lax.dot_general(wn, e, (((1,), (1,)), ((), ())),
                               preferred_element_type=jnp.float32,
                               precision=lax.Precision.HIGHEST)

    s_pos = score(pn)
    s_neg = score(ngn)
    s_neut = score(ntn)
    s_ass = score(an)

    m_neg = jnp.max(s_neg, axis=1, keepdims=True)
    m_neut = jnp.max(s_neut, axis=1, keepdims=True)
    m_ass = s_ass
    m_np = jnp.maximum(jnp.maximum(m_neg, m_neut), m_ass)

    primary = jnp.sum((s_pos >= m_np).astype(jnp.float32), axis=1,
                      keepdims=True)
    secondary = jnp.where(m_neg >= m_np, 0.0,
                          jnp.where(m_neut >= m_np, 1.0, -10.0))
    tot = primary + secondary

    iota_c = lax.broadcasted_iota(jnp.int32, (_VS, 1), 0).astype(jnp.float32)
    key_c = tot * 128.0 - iota_c
    key_r = jnp.transpose(key_c)
    gt = (key_r > key_c).astype(jnp.float32)
    rank = jnp.sum(gt, axis=1, keepdims=True)
    mask_max = (rank < _K).astype(jnp.float32)
    key2_c = -tot * 128.0 - iota_c
    key2_r = jnp.transpose(key2_c)
    gt2 = (key2_r > key2_c).astype(jnp.float32)
    rank2 = jnp.sum(gt2, axis=1, keepdims=True)
    mask_min = (rank2 < _K).astype(jnp.float32)
    onehot = (key_c >= jnp.max(key_c)).astype(jnp.float32)

    def pool(mask):
        s = jnp.sum(w * mask, axis=0, keepdims=True) / float(_K)
        n = jnp.sqrt(jnp.sum(s * s, axis=1, keepdims=True))
        return s / jnp.maximum(n, 1e-12)

    se_ref[...] = jnp.sum(w * onehot, axis=0, keepdims=True)[None]
    mx_ref[...] = pool(mask_max)[None]
    mn_ref[...] = pool(mask_min)[None]


def _run_reward(words, pos, neg, neut, ass):
    out_shape = tuple(jax.ShapeDtypeStruct((_B, 1, _D), jnp.float32)
                      for _ in range(3))
    spec_bd = pl.BlockSpec((1, 1, _D), lambda b: (b, 0, 0))
    se, mx, mn = pl.pallas_call(
        _reward_body,
        grid=(_B,),
        in_specs=[
            pl.BlockSpec((1, _VS, _D), lambda b: (b, 0, 0)),
            pl.BlockSpec((1, _NPOS, _D), lambda b: (b, 0, 0)),
            pl.BlockSpec((1, _NNEG, _D), lambda b: (b, 0, 0)),
            pl.BlockSpec((1, _NNEUT, _D), lambda b: (b, 0, 0)),
            spec_bd,
        ],
        out_specs=(spec_bd, spec_bd, spec_bd),
        out_shape=out_shape,
    )(words, pos, neg, neut, ass[:, None, :])
    return se[:, 0], mx[:, 0], mn[:, 0]


def kernel(pos_embs, neg_embs, neut_embs, assas_emb, vocab_table,
           W1, b1, W2, b2, W3, b3, W4, b4):
    mon = _run_mlp(pos_embs, neg_embs, neut_embs, assas_emb,
                   W1, b1, W2, b2, W3, b3, W4, b4)
    sims, bm3 = _run_sims(mon, vocab_table)
    bm = bm3.transpose(1, 0, 2).reshape(_B, _VPAD // 128)
    thr = _run_thr(bm)
    words = _run_sc_topk(sims, thr, vocab_table)
    se, mx, mn = _run_reward(words, pos_embs, neg_embs, neut_embs, assas_emb)
    return mon, se, mx, mn
